# baseline (speedup 1.0000x reference)
import jax
import jax.numpy as jnp
from jax import lax
from jax.experimental import pallas as pl
from jax.experimental.pallas import tpu as pltpu
from jax.experimental.pallas import tpu_sc as plsc

N = 10000
E = 160000
D = 256
C = 64
NC = 2
NS = 16
K = 128
CH = 80
G = 16
EPAD = NS * CH * K
NACC = 10112
RPT = NACC // NS
R = 400

_MESH = plsc.VectorSubcoreMesh(core_axis_name="c", subcore_axis_name="s",
                               num_cores=NC, num_subcores=NS)


def _zero_rows(buf, nrows, width):
    def zrow(r, carry):
        for kk in range(width // 16):
            buf[0, r, pl.ds(kk * 16, 16)] = jnp.zeros((16,), jnp.float32)
        return carry
    lax.fori_loop(0, nrows, zrow, 0)


def _zero_acc_slice(zsrc, acc, base):
    off = 0
    for sz in (128, 128, 128, 128, RPT - 512):
        pltpu.sync_copy(zsrc.at[pl.ds(0, sz)], acc.at[pl.ds(base + off, sz)])
        off += sz


def _make_sc_agg(width):
    scratch = {
        "acc": pltpu.VMEM_SHARED((NACC, width), jnp.float32),
        "src_v": pltpu.VMEM((G, K), jnp.int32),
        "dst_v": pltpu.VMEM((G, K), jnp.int32),
        "rowbuf": pltpu.VMEM((2, K, width), jnp.float32),
        "sem0": pltpu.SemaphoreType.DMA,
        "sem1": pltpu.SemaphoreType.DMA,
        "ssem0": pltpu.SemaphoreType.DMA,
        "ssem1": pltpu.SemaphoreType.DMA,
    }

    def body(table, src2, dstp, out, *, acc, src_v, dst_v, rowbuf, sem0, sem1,
             ssem0, ssem1):
        c = lax.axis_index("c")
        s = lax.axis_index("s")
        sems = (sem0, sem1)
        ssems = (ssem0, ssem1)

        _zero_rows(rowbuf, K, width)
        base = s * RPT
        _zero_acc_slice(rowbuf.at[0], acc, base)
        plsc.subcore_barrier()

        def gather(j, b):
            pltpu.async_copy(table.at[src_v.at[j]], rowbuf.at[b], sems[b])

        def group_body(g, carry):
            pltpu.sync_copy(src2.at[c, s, pl.ds(g * G, G)], src_v)
            pltpu.sync_copy(dstp.at[s, pl.ds(g * G, G)], dst_v)
            gather(0, 0)
            gather(1, 1)

            def chunk_body(jj, carry2):
                for b in range(2):
                    j = jj * 2 + b
                    pltpu.make_async_copy(table.at[src_v.at[j]],
                                          rowbuf.at[b], sems[b]).wait()
                    cp = pltpu.async_copy(rowbuf.at[b], acc.at[dst_v.at[j]],
                                          ssems[b], add=True)
                    cp.wait()
                    nxt = j + 2

                    @pl.when(nxt < G)
                    def _():
                        gather(nxt, b)
                return carry2
            lax.fori_loop(0, G // 2, chunk_body, 0)
            return carry
        lax.fori_loop(0, CH // G, group_body, 0)

        plsc.subcore_barrier()
        pltpu.sync_copy(acc.at[pl.ds(base, RPT)], out.at[c, pl.ds(base, RPT)])

    return pl.kernel(
        body,
        out_type=[jax.ShapeDtypeStruct((NC, NACC, width), jnp.float32)],
        mesh=_MESH,
        scratch_types=scratch,
        compiler_params=pltpu.CompilerParams(
            use_tc_tiling_on_sc=(width % 128 == 0)),
        name=f"sc_segsum_w{width}",
    )


def _make_sc_deg():
    CHD = CH // NC
    scratch = {
        "dacc": pltpu.VMEM_SHARED((NACC, 16), jnp.float32),
        "dst_v": pltpu.VMEM((CHD, K), jnp.int32),
        "ones_v": pltpu.VMEM((K, 16), jnp.float32),
        "z16": pltpu.VMEM((2, K, 16), jnp.float32),
    }

    def body(dstp, out, *, dacc, dst_v, ones_v, z16):
        c = lax.axis_index("c")
        s = lax.axis_index("s")

        patt = jnp.where(lax.iota(jnp.int32, 16) == 0, 1.0, 0.0)

        def irow(r, carry):
            ones_v[r, :] = patt
            z16[0, r, :] = jnp.zeros((16,), jnp.float32)
            return carry
        lax.fori_loop(0, K, irow, 0)

        base = s * RPT
        _zero_acc_slice(z16.at[0], dacc, base)
        pltpu.sync_copy(dstp.at[s, pl.ds(c * CHD, CHD)], dst_v)
        plsc.subcore_barrier()

        def chunk_body(j, carry):
            pltpu.sync_copy(ones_v, dacc.at[dst_v.at[j]], add=True)
            return carry
        lax.fori_loop(0, CHD, chunk_body, 0)

        plsc.subcore_barrier()
        pltpu.sync_copy(dacc.at[pl.ds(base, RPT)], out.at[c, pl.ds(base, RPT)])

    return pl.kernel(
        body,
        out_type=[jax.ShapeDtypeStruct((NC, NACC, 16), jnp.float32)],
        mesh=_MESH,
        scratch_types=scratch,
        compiler_params=pltpu.CompilerParams(use_tc_tiling_on_sc=False),
        name="sc_degree",
    )


_sc_agg_d = _make_sc_agg(D // 2)
_sc_agg_c = _make_sc_agg(C // 2)
_sc_deg = _make_sc_deg()


def _scale_from_deg(deg_blk):
    deg = deg_blk[0, :, 0:1] + deg_blk[1, :, 0:1]
    return 1.0 / jnp.maximum(deg, 1.0)


def _tc1_body(f_ref, s_ref, deg_ref, w_ref, b_ref, o_ref):
    scale = _scale_from_deg(deg_ref[...])
    agg = jnp.concatenate([s_ref[0], s_ref[1]], axis=1)
    rst = f_ref[...] + agg * scale
    h = jnp.dot(rst, w_ref[...], preferred_element_type=jnp.float32) + b_ref[...]
    h = jnp.maximum(h, 0.0)
    o_ref[0] = h[:, : D // 2]
    o_ref[1] = h[:, D // 2:]


def _tc2_body(h_ref, s_ref, deg_ref, w2_ref, b2_ref, w3_ref, o_ref):
    scale = _scale_from_deg(deg_ref[...])
    h1 = jnp.concatenate([h_ref[0], h_ref[1]], axis=1)
    agg = jnp.concatenate([s_ref[0], s_ref[1]], axis=1)
    rst = h1 + agg * scale
    h2 = jnp.dot(rst, w2_ref[...], preferred_element_type=jnp.float32) + b2_ref[...]
    h2 = jnp.maximum(h2, 0.0)
    y3 = jnp.dot(h2, w3_ref[...], preferred_element_type=jnp.float32)
    o_ref[0] = y3[:, : C // 2]
    o_ref[1] = y3[:, C // 2:]


def _tc3_body(y_ref, s_ref, deg_ref, b3_ref, o_ref):
    scale = _scale_from_deg(deg_ref[...])
    y = jnp.concatenate([y_ref[0], y_ref[1]], axis=1)
    agg = jnp.concatenate([s_ref[0], s_ref[1]], axis=1)
    o_ref[...] = y + agg * scale + b3_ref[...]


def _row_spec(shape, third=False):
    if third:
        return pl.BlockSpec(shape, lambda i: (0, i, 0))
    return pl.BlockSpec(shape, lambda i: (i, 0))


_tc1 = pl.pallas_call(
    _tc1_body,
    grid=(N // R,),
    in_specs=[
        _row_spec((R, D)),
        _row_spec((2, R, D // 2), True),
        _row_spec((2, R, 16), True),
        pl.BlockSpec((D, D), lambda i: (0, 0)),
        pl.BlockSpec((1, D), lambda i: (0, 0)),
    ],
    out_specs=_row_spec((2, R, D // 2), True),
    out_shape=jax.ShapeDtypeStruct((2, N, D // 2), jnp.float32),
)

_tc2 = pl.pallas_call(
    _tc2_body,
    grid=(N // R,),
    in_specs=[
        _row_spec((2, R, D // 2), True),
        _row_spec((2, R, D // 2), True),
        _row_spec((2, R, 16), True),
        pl.BlockSpec((D, D), lambda i: (0, 0)),
        pl.BlockSpec((1, D), lambda i: (0, 0)),
        pl.BlockSpec((D, C), lambda i: (0, 0)),
    ],
    out_specs=_row_spec((2, R, C // 2), True),
    out_shape=jax.ShapeDtypeStruct((2, N, C // 2), jnp.float32),
)

_tc3 = pl.pallas_call(
    _tc3_body,
    grid=(N // R,),
    in_specs=[
        _row_spec((2, R, C // 2), True),
        _row_spec((2, R, C // 2), True),
        _row_spec((2, R, 16), True),
        pl.BlockSpec((1, C), lambda i: (0, 0)),
    ],
    out_specs=_row_spec((R, C)),
    out_shape=jax.ShapeDtypeStruct((N, C), jnp.float32),
)


def kernel(features, edge_index, W1, b1, W2, b2, W3, b3):
    perm = jnp.argsort(edge_index[0])
    src = edge_index[0][perm]
    dst = edge_index[1][perm]
    pad = EPAD - E
    srcp = jnp.concatenate(
        [src, jnp.arange(pad, dtype=jnp.int32)]).reshape(NS, CH, K)
    src2 = jnp.stack([srcp, srcp + N])
    dstp = jnp.concatenate([dst, jnp.full((pad,), N, jnp.int32)]).reshape(NS, CH, K)

    f_cat = jnp.concatenate([features[:, : D // 2], features[:, D // 2:]], axis=0)
    b1r = b1.reshape(1, D)
    b2r = b2.reshape(1, D)
    b3r = b3.reshape(1, C)

    (deg,) = _sc_deg(dstp)
    degn = deg[:, :N, :]
    (s1,) = _sc_agg_d(f_cat, src2, dstp)
    h1 = _tc1(features, s1[:, :N, :], degn, W1, b1r)
    (s2,) = _sc_agg_d(h1.reshape(2 * N, D // 2), src2, dstp)
    y3 = _tc2(h1, s2[:, :N, :], degn, W2, b2r, W3)
    (s3,) = _sc_agg_c(y3.reshape(2 * N, C // 2), src2, dstp)
    out = _tc3(y3, s3[:, :N, :], degn, b3r)
    return out


# speedup vs baseline: 1.1829x; 1.1829x over previous
import jax
import jax.numpy as jnp
from jax import lax
from jax.experimental import pallas as pl
from jax.experimental.pallas import tpu as pltpu
from jax.experimental.pallas import tpu_sc as plsc

N = 10000
E = 160000
D = 256
C = 64
NC = 2
NS = 16
K = 128
CH = 80
G = 16
EPAD = NS * CH * K
NACC = 10112
RPT = NACC // NS
R = 400

_MESH = plsc.VectorSubcoreMesh(core_axis_name="c", subcore_axis_name="s",
                               num_cores=NC, num_subcores=NS)


def _zero_rows(buf, nrows, width):
    def zrow(r, carry):
        for kk in range(width // 16):
            buf[0, r, pl.ds(kk * 16, 16)] = jnp.zeros((16,), jnp.float32)
        return carry
    lax.fori_loop(0, nrows, zrow, 0)


def _zero_acc_slice(zsrc, acc, base):
    off = 0
    for sz in (128, 128, 128, 128, RPT - 512):
        pltpu.sync_copy(zsrc.at[pl.ds(0, sz)], acc.at[pl.ds(base + off, sz)])
        off += sz


def _make_sc_agg(width):
    scratch = {
        "acc": pltpu.VMEM_SHARED((NACC, width), jnp.float32),
        "src_v": pltpu.VMEM((G, K), jnp.int32),
        "dst_v": pltpu.VMEM((G, K), jnp.int32),
        "rowbuf": pltpu.VMEM((2, K, width), jnp.float32),
        "sem0": pltpu.SemaphoreType.DMA,
        "sem1": pltpu.SemaphoreType.DMA,
        "ssem0": pltpu.SemaphoreType.DMA,
        "ssem1": pltpu.SemaphoreType.DMA,
    }

    def body(table, src2, dstp, out, *, acc, src_v, dst_v, rowbuf, sem0, sem1,
             ssem0, ssem1):
        c = lax.axis_index("c")
        s = lax.axis_index("s")
        sems = (sem0, sem1)
        ssems = (ssem0, ssem1)

        _zero_rows(rowbuf, K, width)
        base = s * RPT
        _zero_acc_slice(rowbuf.at[0], acc, base)
        plsc.subcore_barrier()

        def gather(j, b):
            pltpu.async_copy(table.at[src_v.at[j]], rowbuf.at[b], sems[b])

        def group_body(g, carry):
            pltpu.sync_copy(src2.at[c, s, pl.ds(g * G, G)], src_v)
            pltpu.sync_copy(dstp.at[s, pl.ds(g * G, G)], dst_v)
            gather(0, 0)
            gather(1, 1)

            def chunk_body(jj, carry2):
                for b in range(2):
                    j = jj * 2 + b
                    pltpu.make_async_copy(table.at[src_v.at[j]],
                                          rowbuf.at[b], sems[b]).wait()
                    cp = pltpu.async_copy(rowbuf.at[b], acc.at[dst_v.at[j]],
                                          ssems[b], add=True)
                    cp.wait()
                    nxt = j + 2

                    @pl.when(nxt < G)
                    def _():
                        gather(nxt, b)
                return carry2
            lax.fori_loop(0, G // 2, chunk_body, 0)
            return carry
        lax.fori_loop(0, CH // G, group_body, 0)

        plsc.subcore_barrier()
        pltpu.sync_copy(acc.at[pl.ds(base, RPT)], out.at[c, pl.ds(base, RPT)])

    return pl.kernel(
        body,
        out_type=[jax.ShapeDtypeStruct((NC, NACC, width), jnp.float32)],
        mesh=_MESH,
        scratch_types=scratch,
        compiler_params=pltpu.CompilerParams(
            use_tc_tiling_on_sc=(width % 128 == 0)),
        name=f"sc_segsum_w{width}",
    )


def _make_sc_deg():
    CHD = CH // NC
    scratch = {
        "dacc": pltpu.VMEM_SHARED((NACC, 16), jnp.float32),
        "dst_v": pltpu.VMEM((CHD, K), jnp.int32),
        "ones_v": pltpu.VMEM((K, 16), jnp.float32),
        "z16": pltpu.VMEM((2, K, 16), jnp.float32),
    }

    def body(dstp, out, *, dacc, dst_v, ones_v, z16):
        c = lax.axis_index("c")
        s = lax.axis_index("s")

        patt = jnp.where(lax.iota(jnp.int32, 16) == 0, 1.0, 0.0)

        def irow(r, carry):
            ones_v[r, :] = patt
            z16[0, r, :] = jnp.zeros((16,), jnp.float32)
            return carry
        lax.fori_loop(0, K, irow, 0)

        base = s * RPT
        _zero_acc_slice(z16.at[0], dacc, base)
        pltpu.sync_copy(dstp.at[s, pl.ds(c * CHD, CHD)], dst_v)
        plsc.subcore_barrier()

        def chunk_body(j, carry):
            pltpu.sync_copy(ones_v, dacc.at[dst_v.at[j]], add=True)
            return carry
        lax.fori_loop(0, CHD, chunk_body, 0)

        plsc.subcore_barrier()
        pltpu.sync_copy(dacc.at[pl.ds(base, RPT)], out.at[c, pl.ds(base, RPT)])

    return pl.kernel(
        body,
        out_type=[jax.ShapeDtypeStruct((NC, NACC, 16), jnp.float32)],
        mesh=_MESH,
        scratch_types=scratch,
        compiler_params=pltpu.CompilerParams(use_tc_tiling_on_sc=False),
        name="sc_degree",
    )


_sc_agg_d = _make_sc_agg(D // 2)
_sc_agg_c = _make_sc_agg(C // 2)
_sc_deg = _make_sc_deg()


def _scale_from_deg(deg_blk):
    deg = deg_blk[0, :, 0:1] + deg_blk[1, :, 0:1]
    return 1.0 / jnp.maximum(deg, 1.0)


def _tc1_body(f_ref, s_ref, deg_ref, w_ref, b_ref, o_ref):
    scale = _scale_from_deg(deg_ref[...])
    agg = jnp.concatenate([s_ref[0], s_ref[1]], axis=1)
    rst = f_ref[...] + agg * scale
    h = jnp.dot(rst, w_ref[...], preferred_element_type=jnp.float32) + b_ref[...]
    h = jnp.maximum(h, 0.0)
    o_ref[0] = h[:, : D // 2]
    o_ref[1] = h[:, D // 2:]


def _tc2_body(h_ref, s_ref, deg_ref, w2_ref, b2_ref, w3_ref, o_ref):
    scale = _scale_from_deg(deg_ref[...])
    h1 = jnp.concatenate([h_ref[0], h_ref[1]], axis=1)
    agg = jnp.concatenate([s_ref[0], s_ref[1]], axis=1)
    rst = h1 + agg * scale
    h2 = jnp.dot(rst, w2_ref[...], preferred_element_type=jnp.float32) + b2_ref[...]
    h2 = jnp.maximum(h2, 0.0)
    y3 = jnp.dot(h2, w3_ref[...], preferred_element_type=jnp.float32)
    o_ref[0] = y3[:, : C // 2]
    o_ref[1] = y3[:, C // 2:]


def _tc3_body(y_ref, s_ref, deg_ref, b3_ref, o_ref):
    scale = _scale_from_deg(deg_ref[...])
    y = jnp.concatenate([y_ref[0], y_ref[1]], axis=1)
    agg = jnp.concatenate([s_ref[0], s_ref[1]], axis=1)
    o_ref[...] = y + agg * scale + b3_ref[...]


def _row_spec(shape, third=False):
    if third:
        return pl.BlockSpec(shape, lambda i: (0, i, 0))
    return pl.BlockSpec(shape, lambda i: (i, 0))


_tc1 = pl.pallas_call(
    _tc1_body,
    grid=(N // R,),
    in_specs=[
        _row_spec((R, D)),
        _row_spec((2, R, D // 2), True),
        _row_spec((2, R, 16), True),
        pl.BlockSpec((D, D), lambda i: (0, 0)),
        pl.BlockSpec((1, D), lambda i: (0, 0)),
    ],
    out_specs=_row_spec((2, R, D // 2), True),
    out_shape=jax.ShapeDtypeStruct((2, N, D // 2), jnp.float32),
)

_tc2 = pl.pallas_call(
    _tc2_body,
    grid=(N // R,),
    in_specs=[
        _row_spec((2, R, D // 2), True),
        _row_spec((2, R, D // 2), True),
        _row_spec((2, R, 16), True),
        pl.BlockSpec((D, D), lambda i: (0, 0)),
        pl.BlockSpec((1, D), lambda i: (0, 0)),
        pl.BlockSpec((D, C), lambda i: (0, 0)),
    ],
    out_specs=_row_spec((2, R, C // 2), True),
    out_shape=jax.ShapeDtypeStruct((2, N, C // 2), jnp.float32),
)

_tc3 = pl.pallas_call(
    _tc3_body,
    grid=(N // R,),
    in_specs=[
        _row_spec((2, R, C // 2), True),
        _row_spec((2, R, C // 2), True),
        _row_spec((2, R, 16), True),
        pl.BlockSpec((1, C), lambda i: (0, 0)),
    ],
    out_specs=_row_spec((R, C)),
    out_shape=jax.ShapeDtypeStruct((N, C), jnp.float32),
)


def kernel(features, edge_index, W1, b1, W2, b2, W3, b3):
    pad = EPAD - E

    def edge_arrays(src, dst):
        srcp = jnp.concatenate(
            [src, jnp.arange(pad, dtype=jnp.int32)]).reshape(NS, CH, K)
        src2 = jnp.stack([srcp, srcp + N])
        dstp = jnp.concatenate(
            [dst, jnp.full((pad,), N, jnp.int32)]).reshape(NS, CH, K)
        return src2, dstp

    src2, dstp = edge_arrays(edge_index[0], edge_index[1])
    src_s, dst_s = lax.sort((edge_index[0], edge_index[1]), num_keys=1)
    src2_s, dstp_s = edge_arrays(src_s, dst_s)

    f_cat = jnp.concatenate([features[:, : D // 2], features[:, D // 2:]], axis=0)
    b1r = b1.reshape(1, D)
    b2r = b2.reshape(1, D)
    b3r = b3.reshape(1, C)

    (deg,) = _sc_deg(dstp)
    degn = deg[:, :N, :]
    (s1,) = _sc_agg_d(f_cat, src2, dstp)
    h1 = _tc1(features, s1[:, :N, :], degn, W1, b1r)
    (s2,) = _sc_agg_d(h1.reshape(2 * N, D // 2), src2_s, dstp_s)
    y3 = _tc2(h1, s2[:, :N, :], degn, W2, b2r, W3)
    (s3,) = _sc_agg_c(y3.reshape(2 * N, C // 2), src2, dstp)
    out = _tc3(y3, s3[:, :N, :], degn, b3r)
    return out


# speedup vs baseline: 1.2291x; 1.0391x over previous
import jax
import jax.numpy as jnp
from jax import lax
from jax.experimental import pallas as pl
from jax.experimental.pallas import tpu as pltpu
from jax.experimental.pallas import tpu_sc as plsc

N = 10000
E = 160000
D = 256
C = 64
NC = 2
NS = 16
K = 128
CH = 80
G = 16
EPAD = NS * CH * K
NACC = 10112
RPT = NACC // NS
R = 1000

_MESH = plsc.VectorSubcoreMesh(core_axis_name="c", subcore_axis_name="s",
                               num_cores=NC, num_subcores=NS)


def _zero_rows(buf, nrows, width):
    def zrow(r, carry):
        for kk in range(width // 16):
            buf[0, r, pl.ds(kk * 16, 16)] = jnp.zeros((16,), jnp.float32)
        return carry
    lax.fori_loop(0, nrows, zrow, 0)


def _zero_acc_slice(zsrc, acc, base):
    off = 0
    for sz in (128, 128, 128, 128, RPT - 512):
        pltpu.sync_copy(zsrc.at[pl.ds(0, sz)], acc.at[pl.ds(base + off, sz)])
        off += sz


def _make_sc_agg(width):
    scratch = {
        "acc": pltpu.VMEM_SHARED((NACC, width), jnp.float32),
        "src_v": pltpu.VMEM((G, K), jnp.int32),
        "dst_v": pltpu.VMEM((G, K), jnp.int32),
        "rowbuf": pltpu.VMEM((2, K, width), jnp.float32),
        "sem0": pltpu.SemaphoreType.DMA,
        "sem1": pltpu.SemaphoreType.DMA,
        "ssem0": pltpu.SemaphoreType.DMA,
        "ssem1": pltpu.SemaphoreType.DMA,
    }

    def body(table, src2, dstp, out, *, acc, src_v, dst_v, rowbuf, sem0, sem1,
             ssem0, ssem1):
        c = lax.axis_index("c")
        s = lax.axis_index("s")
        sems = (sem0, sem1)
        ssems = (ssem0, ssem1)

        _zero_rows(rowbuf, K, width)
        base = s * RPT
        _zero_acc_slice(rowbuf.at[0], acc, base)
        plsc.subcore_barrier()

        def gather(j, b):
            pltpu.async_copy(table.at[src_v.at[j]], rowbuf.at[b], sems[b])

        def group_body(g, carry):
            pltpu.sync_copy(src2.at[c, s, pl.ds(g * G, G)], src_v)
            pltpu.sync_copy(dstp.at[s, pl.ds(g * G, G)], dst_v)
            gather(0, 0)
            gather(1, 1)

            def chunk_body(jj, carry2):
                for b in range(2):
                    j = jj * 2 + b
                    pltpu.make_async_copy(table.at[src_v.at[j]],
                                          rowbuf.at[b], sems[b]).wait()
                    cp = pltpu.async_copy(rowbuf.at[b], acc.at[dst_v.at[j]],
                                          ssems[b], add=True)
                    cp.wait()
                    nxt = j + 2

                    @pl.when(nxt < G)
                    def _():
                        gather(nxt, b)
                return carry2
            lax.fori_loop(0, G // 2, chunk_body, 0)
            return carry
        lax.fori_loop(0, CH // G, group_body, 0)

        plsc.subcore_barrier()
        pltpu.sync_copy(acc.at[pl.ds(base, RPT)], out.at[c, pl.ds(base, RPT)])

    return pl.kernel(
        body,
        out_type=[jax.ShapeDtypeStruct((NC, NACC, width), jnp.float32)],
        mesh=_MESH,
        scratch_types=scratch,
        compiler_params=pltpu.CompilerParams(
            use_tc_tiling_on_sc=(width % 128 == 0)),
        name=f"sc_segsum_w{width}",
    )


def _make_sc_deg():
    CHD = CH // NC
    scratch = {
        "dacc": pltpu.VMEM_SHARED((NACC, 16), jnp.float32),
        "dst_v": pltpu.VMEM((CHD, K), jnp.int32),
        "ones_v": pltpu.VMEM((K, 16), jnp.float32),
        "z16": pltpu.VMEM((2, K, 16), jnp.float32),
    }

    def body(dstp, out, *, dacc, dst_v, ones_v, z16):
        c = lax.axis_index("c")
        s = lax.axis_index("s")

        patt = jnp.where(lax.iota(jnp.int32, 16) == 0, 1.0, 0.0)

        def irow(r, carry):
            ones_v[r, :] = patt
            z16[0, r, :] = jnp.zeros((16,), jnp.float32)
            return carry
        lax.fori_loop(0, K, irow, 0)

        base = s * RPT
        _zero_acc_slice(z16.at[0], dacc, base)
        pltpu.sync_copy(dstp.at[s, pl.ds(c * CHD, CHD)], dst_v)
        plsc.subcore_barrier()

        def chunk_body(j, carry):
            pltpu.sync_copy(ones_v, dacc.at[dst_v.at[j]], add=True)
            return carry
        lax.fori_loop(0, CHD, chunk_body, 0)

        plsc.subcore_barrier()
        pltpu.sync_copy(dacc.at[pl.ds(base, RPT)], out.at[c, pl.ds(base, RPT)])

    return pl.kernel(
        body,
        out_type=[jax.ShapeDtypeStruct((NC, NACC, 16), jnp.float32)],
        mesh=_MESH,
        scratch_types=scratch,
        compiler_params=pltpu.CompilerParams(use_tc_tiling_on_sc=False),
        name="sc_degree",
    )


_sc_agg_d = _make_sc_agg(D // 2)
_sc_agg_c = _make_sc_agg(C // 2)
_sc_deg = _make_sc_deg()


def _scale_from_deg(deg_blk):
    deg = deg_blk[0, :, 0:1] + deg_blk[1, :, 0:1]
    return 1.0 / jnp.maximum(deg, 1.0)


def _tc1_body(f_ref, s_ref, deg_ref, w_ref, b_ref, o_ref):
    scale = _scale_from_deg(deg_ref[...])
    agg = jnp.concatenate([s_ref[0], s_ref[1]], axis=1)
    rst = f_ref[...] + agg * scale
    h = jnp.dot(rst, w_ref[...], preferred_element_type=jnp.float32) + b_ref[...]
    h = jnp.maximum(h, 0.0)
    o_ref[0] = h[:, : D // 2]
    o_ref[1] = h[:, D // 2:]


def _tc2_body(h_ref, s_ref, deg_ref, w2_ref, b2_ref, w3_ref, o_ref):
    scale = _scale_from_deg(deg_ref[...])
    h1 = jnp.concatenate([h_ref[0], h_ref[1]], axis=1)
    agg = jnp.concatenate([s_ref[0], s_ref[1]], axis=1)
    rst = h1 + agg * scale
    h2 = jnp.dot(rst, w2_ref[...], preferred_element_type=jnp.float32) + b2_ref[...]
    h2 = jnp.maximum(h2, 0.0)
    y3 = jnp.dot(h2, w3_ref[...], preferred_element_type=jnp.float32)
    o_ref[0] = y3[:, : C // 2]
    o_ref[1] = y3[:, C // 2:]


def _tc3_body(y_ref, s_ref, deg_ref, b3_ref, o_ref):
    scale = _scale_from_deg(deg_ref[...])
    y = jnp.concatenate([y_ref[0], y_ref[1]], axis=1)
    agg = jnp.concatenate([s_ref[0], s_ref[1]], axis=1)
    o_ref[...] = y + agg * scale + b3_ref[...]


def _row_spec(shape, third=False):
    if third:
        return pl.BlockSpec(shape, lambda i: (0, i, 0))
    return pl.BlockSpec(shape, lambda i: (i, 0))


_tc1 = pl.pallas_call(
    _tc1_body,
    grid=(N // R,),
    in_specs=[
        _row_spec((R, D)),
        _row_spec((2, R, D // 2), True),
        _row_spec((2, R, 16), True),
        pl.BlockSpec((D, D), lambda i: (0, 0)),
        pl.BlockSpec((1, D), lambda i: (0, 0)),
    ],
    out_specs=_row_spec((2, R, D // 2), True),
    out_shape=jax.ShapeDtypeStruct((2, N, D // 2), jnp.float32),
)

_tc2 = pl.pallas_call(
    _tc2_body,
    grid=(N // R,),
    in_specs=[
        _row_spec((2, R, D // 2), True),
        _row_spec((2, R, D // 2), True),
        _row_spec((2, R, 16), True),
        pl.BlockSpec((D, D), lambda i: (0, 0)),
        pl.BlockSpec((1, D), lambda i: (0, 0)),
        pl.BlockSpec((D, C), lambda i: (0, 0)),
    ],
    out_specs=_row_spec((2, R, C // 2), True),
    out_shape=jax.ShapeDtypeStruct((2, N, C // 2), jnp.float32),
)

_tc3 = pl.pallas_call(
    _tc3_body,
    grid=(N // R,),
    in_specs=[
        _row_spec((2, R, C // 2), True),
        _row_spec((2, R, C // 2), True),
        _row_spec((2, R, 16), True),
        pl.BlockSpec((1, C), lambda i: (0, 0)),
    ],
    out_specs=_row_spec((R, C)),
    out_shape=jax.ShapeDtypeStruct((N, C), jnp.float32),
)


def kernel(features, edge_index, W1, b1, W2, b2, W3, b3):
    pad = EPAD - E

    def edge_arrays(src, dst):
        srcp = jnp.concatenate(
            [src, jnp.arange(pad, dtype=jnp.int32)]).reshape(NS, CH, K)
        src2 = jnp.stack([srcp, srcp + N])
        dstp = jnp.concatenate(
            [dst, jnp.full((pad,), N, jnp.int32)]).reshape(NS, CH, K)
        return src2, dstp

    src2, dstp = edge_arrays(edge_index[0], edge_index[1])
    src_s, dst_s = lax.sort((edge_index[0], edge_index[1]), num_keys=1)
    src2_s, dstp_s = edge_arrays(src_s, dst_s)

    f_cat = jnp.concatenate([features[:, : D // 2], features[:, D // 2:]], axis=0)
    b1r = b1.reshape(1, D)
    b2r = b2.reshape(1, D)
    b3r = b3.reshape(1, C)

    (deg,) = _sc_deg(dstp)
    degn = deg[:, :N, :]
    (s1,) = _sc_agg_d(f_cat, src2, dstp)
    h1 = _tc1(features, s1[:, :N, :], degn, W1, b1r)
    (s2,) = _sc_agg_d(h1.reshape(2 * N, D // 2), src2_s, dstp_s)
    y3 = _tc2(h1, s2[:, :N, :], degn, W2, b2r, W3)
    (s3,) = _sc_agg_c(y3.reshape(2 * N, C // 2), src2, dstp)
    out = _tc3(y3, s3[:, :N, :], degn, b3r)
    return out
